# baseline (speedup 1.0000x reference)
import functools

import jax
import jax.numpy as jnp
from jax import lax
from jax.experimental import pallas as pl
from jax.experimental.pallas import tpu as pltpu
from jax.experimental.pallas import tpu_sc as plsc

_N = 10000
_E = 160000
_D = 256
_H = 4
_C = 256
_B = 64
_L = 2
_HC = _H * _C

_EP = _E + _N
_WIN = 512
_WBUF = _WIN + 32
_EPAD = _EP + _WBUF
_R = 40
_NCH = _N // _R
_OPAD = 272

_ROW_BLK = 1024
_NPAD = 10240


def _mm_body(x_ref, w_ref, o_ref):
    o_ref[...] = jnp.dot(x_ref[...], w_ref[...],
                         preferred_element_type=jnp.float32)


def _project(x, w):
    n, k = x.shape
    m = w.shape[1]
    xp = jnp.pad(x, ((0, _NPAD - n), (0, 0)))
    out = pl.pallas_call(
        _mm_body,
        grid=(_NPAD // _ROW_BLK,),
        in_specs=[
            pl.BlockSpec((_ROW_BLK, k), lambda i: (i, 0)),
            pl.BlockSpec((k, m), lambda i: (0, 0)),
        ],
        out_specs=pl.BlockSpec((_ROW_BLK, m), lambda i: (i, 0)),
        out_shape=jax.ShapeDtypeStruct((_NPAD, m), jnp.float32),
    )(xp, w)
    return out[:n]


def _sc_edge(xl, xr, srcs, dsts, offs, att_flat):
    info = plsc.get_sparse_core_info()
    nw = info.num_cores * info.num_subcores
    scmesh = plsc.VectorSubcoreMesh(core_axis_name="c", subcore_axis_name="s")

    @functools.partial(
        pl.kernel,
        mesh=scmesh,
        compiler_params=pltpu.CompilerParams(use_tc_tiling_on_sc=False,
                                             needs_layout_passes=False),
        out_type=[
            jax.ShapeDtypeStruct((_N, _HC), jnp.float32),
            jax.ShapeDtypeStruct((_N, 16), jnp.float32),
        ],
        scratch_types=[
            pltpu.VMEM((_R, _HC), jnp.float32),
            pltpu.VMEM((_R, _HC), jnp.float32),
            pltpu.VMEM((_R, 16), jnp.float32),
            pltpu.VMEM((_WBUF,), jnp.int32),
            pltpu.VMEM((_WBUF,), jnp.int32),
            pltpu.VMEM((16, _HC), jnp.float32),
            pltpu.VMEM((16, _HC), jnp.float32),
            pltpu.VMEM((16,), jnp.int32),
            pltpu.VMEM((16,), jnp.int32),
            pltpu.VMEM((_HC,), jnp.float32),
            pltpu.VMEM((_OPAD,), jnp.int32),
            pltpu.SemaphoreType.DMA,
            pltpu.SemaphoreType.DMA,
        ],
    )
    def body(xl_h, xr_h, src_h, dst_h, off_h, att_h, out_h, den_h,
             xr_buf, out_buf, den_buf, srcw, dstw, grows0, grows1,
             gidx0, gidx1, attv, offsv, sem0, sem1):
        wid = lax.axis_index("s") * info.num_cores + lax.axis_index("c")
        pltpu.sync_copy(off_h, offsv)
        pltpu.sync_copy(att_h, attv)
        cols = lax.iota(jnp.int32, 16)
        zero16 = jnp.zeros((16,), jnp.float32)

        def issue(b, gidx_b, grows_b, sem_b):
            src_v = srcw[pl.ds(b * 16, 16)]
            gidx_b[...] = jnp.clip(src_v, 0, _N - 1)
            pltpu.make_async_copy(xl_h.at[gidx_b], grows_b, sem_b).start()

        def drain(gidx_b, grows_b, sem_b):
            pltpu.make_async_copy(xl_h.at[gidx_b], grows_b, sem_b).wait()

        def chunk_fn(chunk):
            d0 = chunk * _R
            ov = offsv[pl.ds(chunk, 16)]
            e0 = ov[0]
            e1 = ov[1]
            pltpu.sync_copy(xr_h.at[pl.ds(d0, _R)], xr_buf)

            def zrow(r, _):
                for cg in range(_HC // 16):
                    out_buf[r, pl.ds(cg * 16, 16)] = zero16
                den_buf[r, :] = zero16
                return 0
            lax.fori_loop(0, _R, zrow, 0)

            w0 = (e0 // 8) * 8
            nwin = (e1 - w0 + _WIN - 1) // _WIN
            e0v = jnp.full((16,), e0, jnp.int32)
            e1v = jnp.full((16,), e1, jnp.int32)

            def process(b, ws, grows_b):
                dst_v = dstw[pl.ds(b * 16, 16)]
                local = jnp.clip(dst_v - d0, 0, _R - 1)
                ebv = jnp.full((16,), ws, jnp.int32) + b * 16

                for h in range(_H):
                    at = [attv[pl.ds((h * 16 + t) * 16, 16)]
                          for t in range(16)]
                    oh = cols == h

                    def edge_fn(j, _, h=h, at=at, oh=oh):
                        rows_j = jnp.take(local,
                                          jnp.full((16,), j, jnp.int32))
                        ev = ebv + j
                        ok = (ev >= e0v) & (ev < e1v)
                        xs = []
                        acc = zero16
                        for t in range(16):
                            cg = h * 16 + t
                            xl_v = grows_b[j, pl.ds(cg * 16, 16)]
                            xr_v = plsc.load_gather(
                                xr_buf, [rows_j, cols + cg * 16])
                            z = xl_v + xr_v
                            elr = (jnp.maximum(z, 0.0)
                                   + 0.2 * jnp.minimum(z, 0.0))
                            acc = acc + elr * at[t]
                            xs.append(xl_v)
                        s = lax.reduce_sum_p.bind(acc, axes=(0,))
                        pv = jnp.exp(jnp.full((16,), s))
                        for t in range(16):
                            cg = h * 16 + t
                            plsc.addupdate_scatter(
                                out_buf, [rows_j, cols + cg * 16],
                                pv * xs[t], mask=ok)
                        plsc.addupdate_scatter(
                            den_buf, [rows_j, cols],
                            jnp.where(oh, pv, 0.0), mask=ok & oh)
                        return 0

                    lax.fori_loop(0, 16, edge_fn, 0)

            def win_fn(w, _):
                ws = w0 + w * _WIN
                pltpu.sync_copy(src_h.at[pl.ds(ws, _WBUF)], srcw)
                pltpu.sync_copy(dst_h.at[pl.ds(ws, _WBUF)], dstw)
                nbat = jnp.minimum((e1 - ws + 15) // 16, _WIN // 16)
                npair = (nbat + 1) // 2
                issue(0, gidx0, grows0, sem0)

                def pair_fn(k, _):
                    b0 = 2 * k
                    drain(gidx0, grows0, sem0)
                    issue(b0 + 1, gidx1, grows1, sem1)
                    lax.cond(b0 < nbat,
                             lambda: (process(b0, ws, grows0), 0)[1],
                             lambda: 0)
                    drain(gidx1, grows1, sem1)
                    issue(b0 + 2, gidx0, grows0, sem0)
                    lax.cond(b0 + 1 < nbat,
                             lambda: (process(b0 + 1, ws, grows1), 0)[1],
                             lambda: 0)
                    return 0

                lax.fori_loop(0, npair, pair_fn, 0)
                drain(gidx0, grows0, sem0)
                return 0

            lax.fori_loop(0, nwin, win_fn, 0)
            pltpu.sync_copy(out_buf, out_h.at[pl.ds(d0, _R)])
            pltpu.sync_copy(den_buf, den_h.at[pl.ds(d0, _R)])
            return 0

        def strided(i, _):
            chunk = wid + i * nw
            return lax.cond(chunk < _NCH, lambda: chunk_fn(chunk), lambda: 0)

        lax.fori_loop(0, (_NCH + nw - 1) // nw, strided, 0)

    return body(xl, xr, srcs, dsts, offs, att_flat)


def _prep_edges(ei):
    loop = jnp.arange(_N, dtype=jnp.int32)
    src = jnp.concatenate([ei[0].astype(jnp.int32), loop])
    dst = jnp.concatenate([ei[1].astype(jnp.int32), loop])
    sd, ss = jax.lax.sort((dst, src), num_keys=1)
    offs = jnp.searchsorted(
        sd, jnp.arange(_NCH + 1, dtype=jnp.int32) * _R).astype(jnp.int32)
    ss = jnp.pad(ss, (0, _EPAD - _EP))
    sd = jnp.pad(sd, (0, _EPAD - _EP))
    offs = jnp.pad(offs, (0, _OPAD - (_NCH + 1)), constant_values=_EP)
    return ss, sd, offs


_NORM_BLK = 1000


def _norm_body(u1_ref, d1_ref, u2_ref, d2_ref, bs_ref, o_ref):
    total = bs_ref[...].astype(jnp.float32) * jnp.ones(
        (_NORM_BLK, _C), jnp.float32)
    for h in range(_H):
        total += (u1_ref[:, h * _C:(h + 1) * _C]
                  / (d1_ref[:, h][:, None] + 1e-16))
        total += (u2_ref[:, h * _C:(h + 1) * _C]
                  / (d2_ref[:, h][:, None] + 1e-16))
    o_ref[...] = jnp.tanh(total)


def _norm_combine(u1, d1, u2, d2, bias_sum):
    return pl.pallas_call(
        _norm_body,
        grid=(_N // _NORM_BLK,),
        in_specs=[
            pl.BlockSpec((_NORM_BLK, _HC), lambda i: (i, 0)),
            pl.BlockSpec((_NORM_BLK, 16), lambda i: (i, 0)),
            pl.BlockSpec((_NORM_BLK, _HC), lambda i: (i, 0)),
            pl.BlockSpec((_NORM_BLK, 16), lambda i: (i, 0)),
            pl.BlockSpec((1, _C), lambda i: (0, 0)),
        ],
        out_specs=pl.BlockSpec((_NORM_BLK, _C), lambda i: (i, 0)),
        out_shape=jax.ShapeDtypeStruct((_N, _C), jnp.float32),
    )(u1, d1, u2, d2, bias_sum.reshape(1, _C))


def _head_body(hi_ref, hj_ref, bi_ref, bj_ref, wi_ref, bwi_ref, wj_ref,
               bwj_ref, wl_ref, bwl_ref, pi_ref, pj_ref, yi_ref, yj_ref,
               lg_ref):
    seg_i = bi_ref[0, :]
    seg_j = bj_ref[0, :]
    ids = jax.lax.broadcasted_iota(jnp.int32, (_B, _N), 0)
    oh_i = (seg_i[None, :] == ids).astype(jnp.float32)
    oh_j = (seg_j[None, :] == ids).astype(jnp.float32)
    p_i = jnp.tanh(jnp.dot(oh_i, hi_ref[...],
                           preferred_element_type=jnp.float32))
    p_j = jnp.tanh(jnp.dot(oh_j, hj_ref[...],
                           preferred_element_type=jnp.float32))
    pi_ref[...] = p_i
    pj_ref[...] = p_j
    yi_ref[...] = jnp.dot(p_i, wi_ref[...],
                          preferred_element_type=jnp.float32) + bwi_ref[0, :]
    yj_ref[...] = jnp.dot(p_j, wj_ref[...],
                          preferred_element_type=jnp.float32) + bwj_ref[0, :]
    x = p_i + p_j
    lg_ref[...] = jax.nn.sigmoid(
        jnp.dot(x, wl_ref[...], preferred_element_type=jnp.float32)
        + bwl_ref[0, :])


def _pool_heads(hi, hj, batch_i, batch_j, params):
    outs = pl.pallas_call(
        _head_body,
        in_specs=[
            pl.BlockSpec((_N, _C), lambda: (0, 0)),
            pl.BlockSpec((_N, _C), lambda: (0, 0)),
            pl.BlockSpec((1, _N), lambda: (0, 0)),
            pl.BlockSpec((1, _N), lambda: (0, 0)),
            pl.BlockSpec((_C, 32), lambda: (0, 0)),
            pl.BlockSpec((1, 32), lambda: (0, 0)),
            pl.BlockSpec((_C, 32), lambda: (0, 0)),
            pl.BlockSpec((1, 32), lambda: (0, 0)),
            pl.BlockSpec((_C, 1), lambda: (0, 0)),
            pl.BlockSpec((1, 1), lambda: (0, 0)),
        ],
        out_specs=[
            pl.BlockSpec((_B, _C), lambda: (0, 0)),
            pl.BlockSpec((_B, _C), lambda: (0, 0)),
            pl.BlockSpec((_B, 32), lambda: (0, 0)),
            pl.BlockSpec((_B, 32), lambda: (0, 0)),
            pl.BlockSpec((_B, 1), lambda: (0, 0)),
        ],
        out_shape=[
            jax.ShapeDtypeStruct((_B, _C), jnp.float32),
            jax.ShapeDtypeStruct((_B, _C), jnp.float32),
            jax.ShapeDtypeStruct((_B, 32), jnp.float32),
            jax.ShapeDtypeStruct((_B, 32), jnp.float32),
            jax.ShapeDtypeStruct((_B, 1), jnp.float32),
        ],
    )(hi, hj, batch_i.reshape(1, _N).astype(jnp.int32),
      batch_j.reshape(1, _N).astype(jnp.int32),
      params["W_i"], params["b_i"].reshape(1, 32),
      params["W_j"], params["b_j"].reshape(1, 32),
      params["W_lin"], params["b_lin"].reshape(1, 1))
    _, _, yi, yj, lg = outs
    return lg, yi[:, None, :], yj[:, None, :]


def kernel(x_i, x_j, edge_index_ii, edge_index_jj, edge_index_ij,
           edge_index_ji, batch_i, batch_j, params):
    edges = {
        "ii": _prep_edges(edge_index_ii),
        "jj": _prep_edges(edge_index_jj),
        "ij": _prep_edges(edge_index_ij),
        "ji": _prep_edges(edge_index_ji),
    }
    hi, hj = x_i, x_j
    for l in range(_L):
        xl_ii = _project(hi, params[f"Wl_{l}_ii"])
        xr_ii = _project(hi, params[f"Wr_{l}_ii"])
        xl_ji = _project(hj, params[f"Wl_{l}_ji"])
        xr_ji = _project(hi, params[f"Wr_{l}_ji"])
        xl_jj = _project(hj, params[f"Wl_{l}_jj"])
        xr_jj = _project(hj, params[f"Wr_{l}_jj"])
        xl_ij = _project(hi, params[f"Wl_{l}_ij"])
        xr_ij = _project(hj, params[f"Wr_{l}_ij"])

        u_ii, den_ii = _sc_edge(xl_ii, xr_ii, *edges["ii"],
                                params[f"att_{l}_ii"].reshape(_HC))
        u_ji, den_ji = _sc_edge(xl_ji, xr_ji, *edges["ji"],
                                params[f"att_{l}_ji"].reshape(_HC))
        u_jj, den_jj = _sc_edge(xl_jj, xr_jj, *edges["jj"],
                                params[f"att_{l}_jj"].reshape(_HC))
        u_ij, den_ij = _sc_edge(xl_ij, xr_ij, *edges["ij"],
                                params[f"att_{l}_ij"].reshape(_HC))

        bsum_i = (params[f"b_{l}_ii"] + params[f"b_{l}_ji"]
                  ).reshape(_H, _C).sum(axis=0)
        bsum_j = (params[f"b_{l}_jj"] + params[f"b_{l}_ij"]
                  ).reshape(_H, _C).sum(axis=0)
        hi = _norm_combine(u_ii, den_ii, u_ji, den_ji, bsum_i)
        hj = _norm_combine(u_jj, den_jj, u_ij, den_ij, bsum_j)

    return _pool_heads(hi, hj, batch_i, batch_j, params)


# speedup vs baseline: 1.0694x; 1.0694x over previous
import functools

import jax
import jax.numpy as jnp
from jax import lax
from jax.experimental import pallas as pl
from jax.experimental.pallas import tpu as pltpu
from jax.experimental.pallas import tpu_sc as plsc

_N = 10000
_E = 160000
_D = 256
_H = 4
_C = 256
_B = 64
_L = 2
_HC = _H * _C

_EP = _E + _N
_WIN = 512
_WBUF = _WIN + 32
_EPAD = _EP + _WBUF
_R = 40
_NCH = _N // _R
_OPAD = 272

_ROW_BLK = 1024
_NPAD = 10240


def _mm_body(x_ref, w_ref, o_ref):
    o_ref[...] = jnp.dot(x_ref[...], w_ref[...],
                         preferred_element_type=jnp.float32)


def _project(x, w):
    n, k = x.shape
    m = w.shape[1]
    xp = jnp.pad(x, ((0, _NPAD - n), (0, 0)))
    out = pl.pallas_call(
        _mm_body,
        grid=(_NPAD // _ROW_BLK,),
        in_specs=[
            pl.BlockSpec((_ROW_BLK, k), lambda i: (i, 0)),
            pl.BlockSpec((k, m), lambda i: (0, 0)),
        ],
        out_specs=pl.BlockSpec((_ROW_BLK, m), lambda i: (i, 0)),
        out_shape=jax.ShapeDtypeStruct((_NPAD, m), jnp.float32),
    )(xp, w)
    return out[:n]


def _sc_edge(xl, xr, srcs, dsts, offs, att_flat):
    info = plsc.get_sparse_core_info()
    nw = info.num_cores * info.num_subcores
    scmesh = plsc.VectorSubcoreMesh(core_axis_name="c", subcore_axis_name="s")

    @functools.partial(
        pl.kernel,
        mesh=scmesh,
        compiler_params=pltpu.CompilerParams(use_tc_tiling_on_sc=False,
                                             needs_layout_passes=False),
        out_type=[
            jax.ShapeDtypeStruct((_N, _HC), jnp.float32),
            jax.ShapeDtypeStruct((_N, 16), jnp.float32),
        ],
        scratch_types=[
            pltpu.VMEM((_R, _HC), jnp.float32),
            pltpu.VMEM((_R, _HC), jnp.float32),
            pltpu.VMEM((_R, 16), jnp.float32),
            pltpu.VMEM((_WBUF,), jnp.int32),
            pltpu.VMEM((_WBUF,), jnp.int32),
            pltpu.VMEM((16, _HC), jnp.float32),
            pltpu.VMEM((16, _HC), jnp.float32),
            pltpu.VMEM((16,), jnp.int32),
            pltpu.VMEM((16,), jnp.int32),
            pltpu.VMEM((_HC,), jnp.float32),
            pltpu.VMEM((_OPAD,), jnp.int32),
            pltpu.SemaphoreType.DMA,
            pltpu.SemaphoreType.DMA,
        ],
    )
    def body(xl_h, xr_h, src_h, dst_h, off_h, att_h, out_h, den_h,
             xr_buf, out_buf, den_buf, srcw, dstw, grows0, grows1,
             gidx0, gidx1, attv, offsv, sem0, sem1):
        wid = lax.axis_index("s") * info.num_cores + lax.axis_index("c")
        pltpu.sync_copy(off_h, offsv)
        pltpu.sync_copy(att_h, attv)
        cols = lax.iota(jnp.int32, 16)
        zero16 = jnp.zeros((16,), jnp.float32)

        def issue(b, gidx_b, grows_b, sem_b):
            src_v = srcw[pl.ds(b * 16, 16)]
            gidx_b[...] = jnp.clip(src_v, 0, _N - 1)
            pltpu.make_async_copy(xl_h.at[gidx_b], grows_b, sem_b).start()

        def drain(gidx_b, grows_b, sem_b):
            pltpu.make_async_copy(xl_h.at[gidx_b], grows_b, sem_b).wait()

        def chunk_fn(chunk):
            d0 = chunk * _R
            ov = offsv[pl.ds(chunk, 16)]
            e0 = ov[0]
            e1 = ov[1]
            pltpu.sync_copy(xr_h.at[pl.ds(d0, _R)], xr_buf)

            def zrow(r, _):
                for cg in range(_HC // 16):
                    out_buf[r, pl.ds(cg * 16, 16)] = zero16
                den_buf[r, :] = zero16
                return 0
            lax.fori_loop(0, _R, zrow, 0)

            w0 = (e0 // 8) * 8
            nwin = (e1 - w0 + _WIN - 1) // _WIN
            e0v = jnp.full((16,), e0, jnp.int32)
            e1v = jnp.full((16,), e1, jnp.int32)

            def process(b, ws, grows_b):
                dst_v = dstw[pl.ds(b * 16, 16)]
                local = jnp.clip(dst_v - d0, 0, _R - 1)
                ebv = jnp.full((16,), ws, jnp.int32) + b * 16

                def pair_edges(k, _):
                    for jj in range(2):
                        j = 2 * k + jj
                        rows_j = jnp.take(local,
                                          jnp.full((16,), j, jnp.int32))
                        ev = ebv + j
                        ok = (ev >= e0v) & (ev < e1v)
                        denv = zero16
                        for h in range(_H):
                            xs = []
                            acc = zero16
                            for t in range(16):
                                cg = h * 16 + t
                                xl_v = grows_b[j, pl.ds(cg * 16, 16)]
                                xr_v = plsc.load_gather(
                                    xr_buf, [rows_j, cols + cg * 16])
                                at_v = attv[pl.ds(cg * 16, 16)]
                                z = xl_v + xr_v
                                elr = (jnp.maximum(z, 0.0)
                                       + 0.2 * jnp.minimum(z, 0.0))
                                acc = acc + elr * at_v
                                xs.append(xl_v)
                            s = lax.reduce_sum_p.bind(acc, axes=(0,))
                            pv = jnp.exp(jnp.full((16,), s))
                            denv = denv + jnp.where(cols == h, pv, 0.0)
                            for t in range(16):
                                cg = h * 16 + t
                                plsc.addupdate_scatter(
                                    out_buf, [rows_j, cols + cg * 16],
                                    pv * xs[t], mask=ok)
                        plsc.addupdate_scatter(den_buf, [rows_j, cols],
                                               denv, mask=ok & (cols < _H))
                    return 0

                lax.fori_loop(0, 8, pair_edges, 0)

            def win_fn(w, _):
                ws = w0 + w * _WIN
                pltpu.sync_copy(src_h.at[pl.ds(ws, _WBUF)], srcw)
                pltpu.sync_copy(dst_h.at[pl.ds(ws, _WBUF)], dstw)
                nbat = jnp.minimum((e1 - ws + 15) // 16, _WIN // 16)
                npair = (nbat + 1) // 2
                issue(0, gidx0, grows0, sem0)

                def pair_fn(k, _):
                    b0 = 2 * k
                    drain(gidx0, grows0, sem0)
                    issue(b0 + 1, gidx1, grows1, sem1)
                    lax.cond(b0 < nbat,
                             lambda: (process(b0, ws, grows0), 0)[1],
                             lambda: 0)
                    drain(gidx1, grows1, sem1)
                    issue(b0 + 2, gidx0, grows0, sem0)
                    lax.cond(b0 + 1 < nbat,
                             lambda: (process(b0 + 1, ws, grows1), 0)[1],
                             lambda: 0)
                    return 0

                lax.fori_loop(0, npair, pair_fn, 0)
                drain(gidx0, grows0, sem0)
                return 0

            lax.fori_loop(0, nwin, win_fn, 0)
            pltpu.sync_copy(out_buf, out_h.at[pl.ds(d0, _R)])
            pltpu.sync_copy(den_buf, den_h.at[pl.ds(d0, _R)])
            return 0

        def strided(i, _):
            chunk = wid + i * nw
            return lax.cond(chunk < _NCH, lambda: chunk_fn(chunk), lambda: 0)

        lax.fori_loop(0, (_NCH + nw - 1) // nw, strided, 0)

    return body(xl, xr, srcs, dsts, offs, att_flat)


def _prep_edges(ei):
    loop = jnp.arange(_N, dtype=jnp.int32)
    src = jnp.concatenate([ei[0].astype(jnp.int32), loop])
    dst = jnp.concatenate([ei[1].astype(jnp.int32), loop])
    sd, ss = jax.lax.sort((dst, src), num_keys=1)
    offs = jnp.searchsorted(
        sd, jnp.arange(_NCH + 1, dtype=jnp.int32) * _R).astype(jnp.int32)
    ss = jnp.pad(ss, (0, _EPAD - _EP))
    sd = jnp.pad(sd, (0, _EPAD - _EP))
    offs = jnp.pad(offs, (0, _OPAD - (_NCH + 1)), constant_values=_EP)
    return ss, sd, offs


_NORM_BLK = 1000


def _norm_body(u1_ref, d1_ref, u2_ref, d2_ref, bs_ref, o_ref):
    total = bs_ref[...].astype(jnp.float32) * jnp.ones(
        (_NORM_BLK, _C), jnp.float32)
    for h in range(_H):
        total += (u1_ref[:, h * _C:(h + 1) * _C]
                  / (d1_ref[:, h][:, None] + 1e-16))
        total += (u2_ref[:, h * _C:(h + 1) * _C]
                  / (d2_ref[:, h][:, None] + 1e-16))
    o_ref[...] = jnp.tanh(total)


def _norm_combine(u1, d1, u2, d2, bias_sum):
    return pl.pallas_call(
        _norm_body,
        grid=(_N // _NORM_BLK,),
        in_specs=[
            pl.BlockSpec((_NORM_BLK, _HC), lambda i: (i, 0)),
            pl.BlockSpec((_NORM_BLK, 16), lambda i: (i, 0)),
            pl.BlockSpec((_NORM_BLK, _HC), lambda i: (i, 0)),
            pl.BlockSpec((_NORM_BLK, 16), lambda i: (i, 0)),
            pl.BlockSpec((1, _C), lambda i: (0, 0)),
        ],
        out_specs=pl.BlockSpec((_NORM_BLK, _C), lambda i: (i, 0)),
        out_shape=jax.ShapeDtypeStruct((_N, _C), jnp.float32),
    )(u1, d1, u2, d2, bias_sum.reshape(1, _C))


def _head_body(hi_ref, hj_ref, bi_ref, bj_ref, wi_ref, bwi_ref, wj_ref,
               bwj_ref, wl_ref, bwl_ref, pi_ref, pj_ref, yi_ref, yj_ref,
               lg_ref):
    seg_i = bi_ref[0, :]
    seg_j = bj_ref[0, :]
    ids = jax.lax.broadcasted_iota(jnp.int32, (_B, _N), 0)
    oh_i = (seg_i[None, :] == ids).astype(jnp.float32)
    oh_j = (seg_j[None, :] == ids).astype(jnp.float32)
    p_i = jnp.tanh(jnp.dot(oh_i, hi_ref[...],
                           preferred_element_type=jnp.float32))
    p_j = jnp.tanh(jnp.dot(oh_j, hj_ref[...],
                           preferred_element_type=jnp.float32))
    pi_ref[...] = p_i
    pj_ref[...] = p_j
    yi_ref[...] = jnp.dot(p_i, wi_ref[...],
                          preferred_element_type=jnp.float32) + bwi_ref[0, :]
    yj_ref[...] = jnp.dot(p_j, wj_ref[...],
                          preferred_element_type=jnp.float32) + bwj_ref[0, :]
    x = p_i + p_j
    lg_ref[...] = jax.nn.sigmoid(
        jnp.dot(x, wl_ref[...], preferred_element_type=jnp.float32)
        + bwl_ref[0, :])


def _pool_heads(hi, hj, batch_i, batch_j, params):
    outs = pl.pallas_call(
        _head_body,
        in_specs=[
            pl.BlockSpec((_N, _C), lambda: (0, 0)),
            pl.BlockSpec((_N, _C), lambda: (0, 0)),
            pl.BlockSpec((1, _N), lambda: (0, 0)),
            pl.BlockSpec((1, _N), lambda: (0, 0)),
            pl.BlockSpec((_C, 32), lambda: (0, 0)),
            pl.BlockSpec((1, 32), lambda: (0, 0)),
            pl.BlockSpec((_C, 32), lambda: (0, 0)),
            pl.BlockSpec((1, 32), lambda: (0, 0)),
            pl.BlockSpec((_C, 1), lambda: (0, 0)),
            pl.BlockSpec((1, 1), lambda: (0, 0)),
        ],
        out_specs=[
            pl.BlockSpec((_B, _C), lambda: (0, 0)),
            pl.BlockSpec((_B, _C), lambda: (0, 0)),
            pl.BlockSpec((_B, 32), lambda: (0, 0)),
            pl.BlockSpec((_B, 32), lambda: (0, 0)),
            pl.BlockSpec((_B, 1), lambda: (0, 0)),
        ],
        out_shape=[
            jax.ShapeDtypeStruct((_B, _C), jnp.float32),
            jax.ShapeDtypeStruct((_B, _C), jnp.float32),
            jax.ShapeDtypeStruct((_B, 32), jnp.float32),
            jax.ShapeDtypeStruct((_B, 32), jnp.float32),
            jax.ShapeDtypeStruct((_B, 1), jnp.float32),
        ],
    )(hi, hj, batch_i.reshape(1, _N).astype(jnp.int32),
      batch_j.reshape(1, _N).astype(jnp.int32),
      params["W_i"], params["b_i"].reshape(1, 32),
      params["W_j"], params["b_j"].reshape(1, 32),
      params["W_lin"], params["b_lin"].reshape(1, 1))
    _, _, yi, yj, lg = outs
    return lg, yi[:, None, :], yj[:, None, :]


def kernel(x_i, x_j, edge_index_ii, edge_index_jj, edge_index_ij,
           edge_index_ji, batch_i, batch_j, params):
    edges = {
        "ii": _prep_edges(edge_index_ii),
        "jj": _prep_edges(edge_index_jj),
        "ij": _prep_edges(edge_index_ij),
        "ji": _prep_edges(edge_index_ji),
    }
    hi, hj = x_i, x_j
    for l in range(_L):
        xl_ii = _project(hi, params[f"Wl_{l}_ii"])
        xr_ii = _project(hi, params[f"Wr_{l}_ii"])
        xl_ji = _project(hj, params[f"Wl_{l}_ji"])
        xr_ji = _project(hi, params[f"Wr_{l}_ji"])
        xl_jj = _project(hj, params[f"Wl_{l}_jj"])
        xr_jj = _project(hj, params[f"Wr_{l}_jj"])
        xl_ij = _project(hi, params[f"Wl_{l}_ij"])
        xr_ij = _project(hj, params[f"Wr_{l}_ij"])

        u_ii, den_ii = _sc_edge(xl_ii, xr_ii, *edges["ii"],
                                params[f"att_{l}_ii"].reshape(_HC))
        u_ji, den_ji = _sc_edge(xl_ji, xr_ji, *edges["ji"],
                                params[f"att_{l}_ji"].reshape(_HC))
        u_jj, den_jj = _sc_edge(xl_jj, xr_jj, *edges["jj"],
                                params[f"att_{l}_jj"].reshape(_HC))
        u_ij, den_ij = _sc_edge(xl_ij, xr_ij, *edges["ij"],
                                params[f"att_{l}_ij"].reshape(_HC))

        bsum_i = (params[f"b_{l}_ii"] + params[f"b_{l}_ji"]
                  ).reshape(_H, _C).sum(axis=0)
        bsum_j = (params[f"b_{l}_jj"] + params[f"b_{l}_ij"]
                  ).reshape(_H, _C).sum(axis=0)
        hi = _norm_combine(u_ii, den_ii, u_ji, den_ji, bsum_i)
        hj = _norm_combine(u_jj, den_jj, u_ij, den_ij, bsum_j)

    return _pool_heads(hi, hj, batch_i, batch_j, params)


# speedup vs baseline: 1.1020x; 1.0305x over previous
import functools

import jax
import jax.numpy as jnp
from jax import lax
from jax.experimental import pallas as pl
from jax.experimental.pallas import tpu as pltpu
from jax.experimental.pallas import tpu_sc as plsc

_N = 10000
_E = 160000
_D = 256
_H = 4
_C = 256
_B = 64
_L = 2
_HC = _H * _C

_EP = _E + _N
_WIN = 512
_WBUF = _WIN + 32
_EPAD = _EP + _WBUF
_R = 40
_NCH = _N // _R
_OPAD = 272

_ROW_BLK = 1024
_NPAD = 10240


def _mm_body(x_ref, w_ref, o_ref):
    o_ref[...] = jnp.dot(x_ref[...], w_ref[...],
                         preferred_element_type=jnp.float32)


def _project(x, w):
    n, k = x.shape
    m = w.shape[1]
    xp = jnp.pad(x, ((0, _NPAD - n), (0, 0)))
    out = pl.pallas_call(
        _mm_body,
        grid=(_NPAD // _ROW_BLK,),
        in_specs=[
            pl.BlockSpec((_ROW_BLK, k), lambda i: (i, 0)),
            pl.BlockSpec((k, m), lambda i: (0, 0)),
        ],
        out_specs=pl.BlockSpec((_ROW_BLK, m), lambda i: (i, 0)),
        out_shape=jax.ShapeDtypeStruct((_NPAD, m), jnp.float32),
    )(xp, w)
    return out[:n]


def _sc_edge(xl, xr, srcs, dsts, offs, att_flat):
    info = plsc.get_sparse_core_info()
    nw = info.num_cores * info.num_subcores
    scmesh = plsc.VectorSubcoreMesh(core_axis_name="c", subcore_axis_name="s")

    @functools.partial(
        pl.kernel,
        mesh=scmesh,
        compiler_params=pltpu.CompilerParams(use_tc_tiling_on_sc=False,
                                             needs_layout_passes=False),
        out_type=[
            jax.ShapeDtypeStruct((_N, _HC), jnp.float32),
            jax.ShapeDtypeStruct((_N, 16), jnp.float32),
        ],
        scratch_types=[
            pltpu.VMEM((_R, _HC), jnp.float32),
            pltpu.VMEM((_R, _HC), jnp.float32),
            pltpu.VMEM((_R, 16), jnp.float32),
            pltpu.VMEM((_WBUF,), jnp.int32),
            pltpu.VMEM((_WBUF,), jnp.int32),
            pltpu.VMEM((16, _HC), jnp.float32),
            pltpu.VMEM((16, _HC), jnp.float32),
            pltpu.VMEM((16,), jnp.int32),
            pltpu.VMEM((16,), jnp.int32),
            pltpu.VMEM((_HC,), jnp.float32),
            pltpu.VMEM((_OPAD,), jnp.int32),
            pltpu.SemaphoreType.DMA,
            pltpu.SemaphoreType.DMA,
        ],
    )
    def body(xl_h, xr_h, src_h, dst_h, off_h, att_h, out_h, den_h,
             xr_buf, out_buf, den_buf, srcw, dstw, grows0, grows1,
             gidx0, gidx1, attv, offsv, sem0, sem1):
        wid = lax.axis_index("s") * info.num_cores + lax.axis_index("c")
        pltpu.sync_copy(off_h, offsv)
        pltpu.sync_copy(att_h, attv)
        cols = lax.iota(jnp.int32, 16)
        zero16 = jnp.zeros((16,), jnp.float32)

        def issue(b, gidx_b, grows_b, sem_b):
            src_v = srcw[pl.ds(b * 16, 16)]
            gidx_b[...] = jnp.clip(src_v, 0, _N - 1)
            pltpu.make_async_copy(xl_h.at[gidx_b], grows_b, sem_b).start()

        def drain(gidx_b, grows_b, sem_b):
            pltpu.make_async_copy(xl_h.at[gidx_b], grows_b, sem_b).wait()

        def chunk_fn(chunk):
            d0 = chunk * _R
            ov = offsv[pl.ds(chunk, 16)]
            e0 = ov[0]
            e1 = ov[1]
            pltpu.sync_copy(xr_h.at[pl.ds(d0, _R)], xr_buf)

            def zrow(r, _):
                for cg in range(_HC // 16):
                    out_buf[r, pl.ds(cg * 16, 16)] = zero16
                den_buf[r, :] = zero16
                return 0
            lax.fori_loop(0, _R, zrow, 0)

            w0 = (e0 // 8) * 8
            nwin = (e1 - w0 + _WIN - 1) // _WIN
            e0v = jnp.full((16,), e0, jnp.int32)
            e1v = jnp.full((16,), e1, jnp.int32)

            def process(b, ws, grows_b):
                dst_v = dstw[pl.ds(b * 16, 16)]
                local = jnp.clip(dst_v - d0, 0, _R - 1)
                ebv = jnp.full((16,), ws, jnp.int32) + b * 16

                def edge_fn(j, _):
                    rows_j = jnp.take(local, jnp.full((16,), j, jnp.int32))
                    ev = ebv + j
                    ok = (ev >= e0v) & (ev < e1v)
                    denv = zero16
                    for h in range(_H):
                        xs = []
                        acc = zero16
                        for t in range(16):
                            cg = h * 16 + t
                            xl_v = grows_b[j, pl.ds(cg * 16, 16)]
                            xr_v = plsc.load_gather(
                                xr_buf, [rows_j, cols + cg * 16])
                            at_v = attv[pl.ds(cg * 16, 16)]
                            z = xl_v + xr_v
                            elr = (jnp.maximum(z, 0.0)
                                   + 0.2 * jnp.minimum(z, 0.0))
                            acc = acc + elr * at_v
                            xs.append(xl_v)
                        s = lax.reduce_sum_p.bind(acc, axes=(0,))
                        pv = jnp.exp(jnp.full((16,), s))
                        denv = denv + jnp.where(cols == h, pv, 0.0)
                        for t in range(16):
                            cg = h * 16 + t
                            plsc.addupdate_scatter(
                                out_buf, [rows_j, cols + cg * 16],
                                pv * xs[t], mask=ok)
                    plsc.addupdate_scatter(den_buf, [rows_j, cols], denv,
                                           mask=ok & (cols < _H))
                    return 0

                lax.fori_loop(0, 16, edge_fn, 0)

            def win_fn(w, _):
                ws = w0 + w * _WIN
                pltpu.sync_copy(src_h.at[pl.ds(ws, _WBUF)], srcw)
                pltpu.sync_copy(dst_h.at[pl.ds(ws, _WBUF)], dstw)
                nbat = jnp.minimum((e1 - ws + 15) // 16, _WIN // 16)
                npair = (nbat + 1) // 2
                issue(0, gidx0, grows0, sem0)

                def pair_fn(k, _):
                    b0 = 2 * k
                    drain(gidx0, grows0, sem0)
                    issue(b0 + 1, gidx1, grows1, sem1)
                    lax.cond(b0 < nbat,
                             lambda: (process(b0, ws, grows0), 0)[1],
                             lambda: 0)
                    drain(gidx1, grows1, sem1)
                    issue(b0 + 2, gidx0, grows0, sem0)
                    lax.cond(b0 + 1 < nbat,
                             lambda: (process(b0 + 1, ws, grows1), 0)[1],
                             lambda: 0)
                    return 0

                lax.fori_loop(0, npair, pair_fn, 0)
                drain(gidx0, grows0, sem0)
                return 0

            lax.fori_loop(0, nwin, win_fn, 0)
            pltpu.sync_copy(out_buf, out_h.at[pl.ds(d0, _R)])
            pltpu.sync_copy(den_buf, den_h.at[pl.ds(d0, _R)])
            return 0

        def strided(i, _):
            chunk = wid + i * nw
            return lax.cond(chunk < _NCH, lambda: chunk_fn(chunk), lambda: 0)

        lax.fori_loop(0, (_NCH + nw - 1) // nw, strided, 0)

    return body(xl, xr, srcs, dsts, offs, att_flat)


def _prep_edges(ei):
    loop = jnp.arange(_N, dtype=jnp.int32)
    src = jnp.concatenate([ei[0].astype(jnp.int32), loop])
    dst = jnp.concatenate([ei[1].astype(jnp.int32), loop])
    sd, ss = jax.lax.sort((dst, src), num_keys=1)
    offs = jnp.searchsorted(
        sd, jnp.arange(_NCH + 1, dtype=jnp.int32) * _R).astype(jnp.int32)
    ss = jnp.pad(ss, (0, _EPAD - _EP))
    sd = jnp.pad(sd, (0, _EPAD - _EP))
    offs = jnp.pad(offs, (0, _OPAD - (_NCH + 1)), constant_values=_EP)
    return ss, sd, offs


_NORM_BLK = 1000


def _norm_body(u1_ref, d1_ref, u2_ref, d2_ref, bs_ref, o_ref):
    total = bs_ref[...].astype(jnp.float32) * jnp.ones(
        (_NORM_BLK, _C), jnp.float32)
    for h in range(_H):
        total += (u1_ref[:, h * _C:(h + 1) * _C]
                  / (d1_ref[:, h][:, None] + 1e-16))
        total += (u2_ref[:, h * _C:(h + 1) * _C]
                  / (d2_ref[:, h][:, None] + 1e-16))
    o_ref[...] = jnp.tanh(total)


def _norm_combine(u1, d1, u2, d2, bias_sum):
    return pl.pallas_call(
        _norm_body,
        grid=(_N // _NORM_BLK,),
        in_specs=[
            pl.BlockSpec((_NORM_BLK, _HC), lambda i: (i, 0)),
            pl.BlockSpec((_NORM_BLK, 16), lambda i: (i, 0)),
            pl.BlockSpec((_NORM_BLK, _HC), lambda i: (i, 0)),
            pl.BlockSpec((_NORM_BLK, 16), lambda i: (i, 0)),
            pl.BlockSpec((1, _C), lambda i: (0, 0)),
        ],
        out_specs=pl.BlockSpec((_NORM_BLK, _C), lambda i: (i, 0)),
        out_shape=jax.ShapeDtypeStruct((_N, _C), jnp.float32),
    )(u1, d1, u2, d2, bias_sum.reshape(1, _C))


def _head_body(hi_ref, hj_ref, bi_ref, bj_ref, wi_ref, bwi_ref, wj_ref,
               bwj_ref, wl_ref, bwl_ref, pi_ref, pj_ref, yi_ref, yj_ref,
               lg_ref):
    seg_i = bi_ref[0, :]
    seg_j = bj_ref[0, :]
    ids = jax.lax.broadcasted_iota(jnp.int32, (_B, _N), 0)
    oh_i = (seg_i[None, :] == ids).astype(jnp.float32)
    oh_j = (seg_j[None, :] == ids).astype(jnp.float32)
    p_i = jnp.tanh(jnp.dot(oh_i, hi_ref[...],
                           preferred_element_type=jnp.float32))
    p_j = jnp.tanh(jnp.dot(oh_j, hj_ref[...],
                           preferred_element_type=jnp.float32))
    pi_ref[...] = p_i
    pj_ref[...] = p_j
    yi_ref[...] = jnp.dot(p_i, wi_ref[...],
                          preferred_element_type=jnp.float32) + bwi_ref[0, :]
    yj_ref[...] = jnp.dot(p_j, wj_ref[...],
                          preferred_element_type=jnp.float32) + bwj_ref[0, :]
    x = p_i + p_j
    lg_ref[...] = jax.nn.sigmoid(
        jnp.dot(x, wl_ref[...], preferred_element_type=jnp.float32)
        + bwl_ref[0, :])


def _pool_heads(hi, hj, batch_i, batch_j, params):
    outs = pl.pallas_call(
        _head_body,
        in_specs=[
            pl.BlockSpec((_N, _C), lambda: (0, 0)),
            pl.BlockSpec((_N, _C), lambda: (0, 0)),
            pl.BlockSpec((1, _N), lambda: (0, 0)),
            pl.BlockSpec((1, _N), lambda: (0, 0)),
            pl.BlockSpec((_C, 32), lambda: (0, 0)),
            pl.BlockSpec((1, 32), lambda: (0, 0)),
            pl.BlockSpec((_C, 32), lambda: (0, 0)),
            pl.BlockSpec((1, 32), lambda: (0, 0)),
            pl.BlockSpec((_C, 1), lambda: (0, 0)),
            pl.BlockSpec((1, 1), lambda: (0, 0)),
        ],
        out_specs=[
            pl.BlockSpec((_B, _C), lambda: (0, 0)),
            pl.BlockSpec((_B, _C), lambda: (0, 0)),
            pl.BlockSpec((_B, 32), lambda: (0, 0)),
            pl.BlockSpec((_B, 32), lambda: (0, 0)),
            pl.BlockSpec((_B, 1), lambda: (0, 0)),
        ],
        out_shape=[
            jax.ShapeDtypeStruct((_B, _C), jnp.float32),
            jax.ShapeDtypeStruct((_B, _C), jnp.float32),
            jax.ShapeDtypeStruct((_B, 32), jnp.float32),
            jax.ShapeDtypeStruct((_B, 32), jnp.float32),
            jax.ShapeDtypeStruct((_B, 1), jnp.float32),
        ],
    )(hi, hj, batch_i.reshape(1, _N).astype(jnp.int32),
      batch_j.reshape(1, _N).astype(jnp.int32),
      params["W_i"], params["b_i"].reshape(1, 32),
      params["W_j"], params["b_j"].reshape(1, 32),
      params["W_lin"], params["b_lin"].reshape(1, 1))
    _, _, yi, yj, lg = outs
    return lg, yi[:, None, :], yj[:, None, :]


def kernel(x_i, x_j, edge_index_ii, edge_index_jj, edge_index_ij,
           edge_index_ji, batch_i, batch_j, params):
    edges = {
        "ii": _prep_edges(edge_index_ii),
        "jj": _prep_edges(edge_index_jj),
        "ij": _prep_edges(edge_index_ij),
        "ji": _prep_edges(edge_index_ji),
    }
    hi, hj = x_i, x_j
    for l in range(_L):
        xl_ii = _project(hi, params[f"Wl_{l}_ii"])
        xr_ii = _project(hi, params[f"Wr_{l}_ii"])
        xl_ji = _project(hj, params[f"Wl_{l}_ji"])
        xr_ji = _project(hi, params[f"Wr_{l}_ji"])
        xl_jj = _project(hj, params[f"Wl_{l}_jj"])
        xr_jj = _project(hj, params[f"Wr_{l}_jj"])
        xl_ij = _project(hi, params[f"Wl_{l}_ij"])
        xr_ij = _project(hj, params[f"Wr_{l}_ij"])

        u_ii, den_ii = _sc_edge(xl_ii, xr_ii, *edges["ii"],
                                params[f"att_{l}_ii"].reshape(_HC))
        u_ji, den_ji = _sc_edge(xl_ji, xr_ji, *edges["ji"],
                                params[f"att_{l}_ji"].reshape(_HC))
        u_jj, den_jj = _sc_edge(xl_jj, xr_jj, *edges["jj"],
                                params[f"att_{l}_jj"].reshape(_HC))
        u_ij, den_ij = _sc_edge(xl_ij, xr_ij, *edges["ij"],
                                params[f"att_{l}_ij"].reshape(_HC))

        bsum_i = (params[f"b_{l}_ii"] + params[f"b_{l}_ji"]
                  ).reshape(_H, _C).sum(axis=0)
        bsum_j = (params[f"b_{l}_jj"] + params[f"b_{l}_ij"]
                  ).reshape(_H, _C).sum(axis=0)
        hi = _norm_combine(u_ii, den_ii, u_ji, den_ji, bsum_i)
        hj = _norm_combine(u_jj, den_jj, u_ij, den_ij, bsum_j)

    return _pool_heads(hi, hj, batch_i, batch_j, params)
